# baseline (speedup 1.0000x reference)
import functools

import jax
import jax.numpy as jnp
from jax import lax
from jax.experimental import pallas as pl
from jax.experimental.pallas import tpu as pltpu
from jax.experimental.pallas import tpu_sc as plsc

N, E, H, G = 10000, 320000, 128, 16
NP = 10240
HH = H // 2
NC, NS = 2, 16
EPW = E // NS
B = 80
NB = EPW // B
RPT = NP // NS
ZR = 128

_LOG2 = 0.6931471805599453
_PI = 3.141592653589793
_CUT = 10.0


def _ssp(v):
    return jnp.maximum(v, 0.0) + jnp.log1p(jnp.exp(-jnp.abs(v))) - _LOG2


_BE = 2560


def _filter_body(eat, el, w0, b0, w2, b2, out):
    a = eat[...]
    z = lax.dot_general(a, w0[...], (((0,), (0,)), ((), ())),
                        preferred_element_type=jnp.float32) + b0[...]
    w = jnp.dot(_ssp(z), w2[...], preferred_element_type=jnp.float32) + b2[...]
    l = el[0]
    env = 0.5 * (jnp.cos(l * (_PI / _CUT)) + 1.0)
    env = jnp.where((l <= _CUT) & (l >= 0.0), env, 0.0)
    w3 = w.reshape(_BE // 128, 128, H) * env[:, :, None]
    out[...] = w3.reshape(_BE, H)


def _edge_filter(edge_attr_t, el128, w0t, b0, w2t, b2, nblk, blk_off):
    be = _BE
    return pl.pallas_call(
        _filter_body,
        grid=(nblk,),
        in_specs=[
            pl.BlockSpec((G, be), lambda i: (0, blk_off + i)),
            pl.BlockSpec((1, be // 128, 128), lambda i: (blk_off + i, 0, 0)),
            pl.BlockSpec((G, H), lambda i: (0, 0)),
            pl.BlockSpec((1, H), lambda i: (0, 0)),
            pl.BlockSpec((H, H), lambda i: (0, 0)),
            pl.BlockSpec((1, H), lambda i: (0, 0)),
        ],
        out_specs=pl.BlockSpec((be, H), lambda i: (i, 0)),
        out_shape=jax.ShapeDtypeStruct((nblk * be, H), jnp.float32),
    )(edge_attr_t, el128, w0t, b0, w2t, b2)


def _lin1_body(xr, wr, out):
    h = jnp.dot(xr[...], wr[...], preferred_element_type=jnp.float32)
    out[pl.ds(0, N), :] = h[:, :HH]
    out[pl.ds(NP, N), :] = h[:, HH:]


def _lin1(x, w1t):
    return pl.pallas_call(
        _lin1_body,
        out_shape=jax.ShapeDtypeStruct((2 * NP, HH), jnp.float32),
    )(x, w1t)


def _tail_body(p0, p1, p2, p3, w2l, b2l, wl, bl, out):
    acc = p0[...] + p1[...] + p2[...] + p3[...]
    r = jnp.concatenate([acc[0, :, :HH], acc[1, :, :HH]], axis=1)
    t = _ssp(jnp.dot(r, w2l[...], preferred_element_type=jnp.float32) + b2l[...])
    out[...] = jnp.dot(t, wl[...], preferred_element_type=jnp.float32) + bl[...]


def _tail(parts, w2lt, b2l, wlt, bl):
    bn = 2000
    pspec = pl.BlockSpec((NC, bn, H), lambda i: (0, i, 0))
    return pl.pallas_call(
        _tail_body,
        grid=(N // bn,),
        in_specs=[
            pspec, pspec, pspec, pspec,
            pl.BlockSpec((H, H), lambda i: (0, 0)),
            pl.BlockSpec((1, H), lambda i: (0, 0)),
            pl.BlockSpec((H, H), lambda i: (0, 0)),
            pl.BlockSpec((1, H), lambda i: (0, 0)),
        ],
        out_specs=pl.BlockSpec((bn, H), lambda i: (i, 0)),
        out_shape=jax.ShapeDtypeStruct((N, H), jnp.float32),
    )(*parts, w2lt, b2l, wlt, bl)


@functools.cache
def _sc_msgpass_call(nb, chunk_base):
    epw = nb * B

    def _sc_msgpass(h_hbm, w_hbm, src_hbm, dst_hbm, out_hbm,
                    src_v, dst_v, gbuf_a, gbuf_b, gbuf_c, gbuf_d,
                    wbuf_a, wbuf_b, wbuf_c, wbuf_d,
                    sbuf_a, sbuf_b, sbuf_c, sbuf_d, zbuf, acc,
                    sem_ga, sem_gb, sem_gc, sem_gd,
                    sem_wa, sem_wb, sem_wc, sem_wd,
                    sem_sa, sem_sb, sem_sc, sem_sd):
        c = lax.axis_index("c")
        s = lax.axis_index("s")

        pltpu.sync_copy(src_hbm.at[s], src_v)
        pltpu.sync_copy(dst_hbm.at[s], dst_v)

        coff = jnp.full((16,), c * NP, jnp.int32)

        def orow(r, carry):
            for k in range(B // 16):
                sl = pl.ds(k * 16, 16)
                src_v[r, sl] = src_v[r, sl] + coff
            return carry

        lax.fori_loop(0, nb, orow, 0)

        zeros = jnp.zeros((16,), jnp.float32)

        def zrow(r, carry):
            for k in range(HH // 16):
                zbuf[r, pl.ds(k * 16, 16)] = zeros
            return carry

        lax.fori_loop(0, ZR, zrow, 0)

        def zcopy(t, carry):
            pltpu.sync_copy(zbuf, acc.at[pl.ds(s * RPT + t * ZR, ZR)])
            return carry

        lax.fori_loop(0, RPT // ZR, zcopy, 0)
        plsc.subcore_barrier()

        ebase = chunk_base + s * epw

        def fetch(j, gbuf, wbuf, sem_g, sem_w):
            pltpu.async_copy(h_hbm.at[src_v.at[j]], gbuf, sem_g)
            pltpu.async_copy(
                w_hbm.at[pl.ds(ebase + j * B, B), pl.ds(c * HH, HH)],
                wbuf, sem_w)

        def wait_fetch(j, gbuf, wbuf, sem_g, sem_w):
            pltpu.make_async_copy(h_hbm.at[src_v.at[j]], gbuf, sem_g).wait()
            pltpu.make_async_copy(
                w_hbm.at[pl.ds(ebase + j * B, B), pl.ds(c * HH, HH)],
                wbuf, sem_w).wait()

        def mul(gbuf, wbuf, sbuf):
            def mrow(r4, inner):
                for dr in range(4):
                    r = r4 * 4 + dr
                    for q in range(HH // 32):
                        hv = gbuf[r, pl.ds(q * 16, 16)]
                        hb = plsc.bitcast(hv, jnp.bfloat16)
                        he, ho = plsc.unpack(
                            hb, format=plsc.PackFormat.INTERLEAVED)
                        sl_e = pl.ds(q * 32, 16)
                        sl_o = pl.ds(q * 32 + 16, 16)
                        sbuf[r, sl_e] = he * wbuf[r, sl_e]
                        sbuf[r, sl_o] = ho * wbuf[r, sl_o]
                return inner
            lax.fori_loop(0, B // 4, mrow, 0)

        def scat(j, gbuf, sem_s):
            pltpu.async_copy(gbuf, acc.at[dst_v.at[j]], sem_s, add=True)

        def wait_scat(j, gbuf, sem_s):
            pltpu.make_async_copy(gbuf, acc.at[dst_v.at[j]], sem_s).wait()

        K = 4
        gbufs = [gbuf_a, gbuf_b, gbuf_c, gbuf_d]
        wbufs = [wbuf_a, wbuf_b, wbuf_c, wbuf_d]
        sbufs = [sbuf_a, sbuf_b, sbuf_c, sbuf_d]
        sgs = [sem_ga, sem_gb, sem_gc, sem_gd]
        sws = [sem_wa, sem_wb, sem_wc, sem_wd]
        sss = [sem_sa, sem_sb, sem_sc, sem_sd]

        for t in range(K - 1):
            fetch(t, gbufs[t], wbufs[t], sgs[t], sws[t])

        ngrp = nb // K
        rem = nb - ngrp * K

        def step(j, t, drain):
            wait_fetch(j, gbufs[t], wbufs[t], sgs[t], sws[t])
            mul(gbufs[t], wbufs[t], sbufs[t])
            scat(j, sbufs[t], sss[t])
            tf = (t - 1) % K
            jf = j + K - 1

            @pl.when(jf < nb)
            def _():
                if drain:
                    wait_scat(j - 1, sbufs[tf], sss[tf])
                fetch(jf, gbufs[tf], wbufs[tf], sgs[tf], sws[tf])

        for t in range(K):
            step(t, t, t > 0)

        def body(g, carry):
            j0 = g * K
            for t in range(K):
                step(j0 + t, t, True)
            return carry

        lax.fori_loop(1, ngrp, body, 0)
        for r in range(rem):
            step(ngrp * K + r, r, True)
        for r in range(K):
            j = nb - K + r
            wait_scat(j, sbufs[(nb - K + r) % K], sss[(nb - K + r) % K])

        plsc.subcore_barrier()
        pltpu.sync_copy(acc.at[pl.ds(s * RPT, RPT)],
                        out_hbm.at[c, pl.ds(s * RPT, RPT), pl.ds(0, HH)])

    mesh = plsc.VectorSubcoreMesh(
        core_axis_name="c", subcore_axis_name="s",
        num_cores=NC, num_subcores=NS)
    return pl.kernel(
        _sc_msgpass,
        out_type=jax.ShapeDtypeStruct((NC, NP, H), jnp.float32),
        mesh=mesh,
        scratch_types=[
            pltpu.VMEM((nb, B), jnp.int32),
            pltpu.VMEM((nb, B), jnp.int32),
            pltpu.VMEM((B, HH // 2), jnp.int32),
            pltpu.VMEM((B, HH // 2), jnp.int32),
            pltpu.VMEM((B, HH // 2), jnp.int32),
            pltpu.VMEM((B, HH // 2), jnp.int32),
            pltpu.VMEM((B, HH), jnp.float32),
            pltpu.VMEM((B, HH), jnp.float32),
            pltpu.VMEM((B, HH), jnp.float32),
            pltpu.VMEM((B, HH), jnp.float32),
            pltpu.VMEM((B, HH), jnp.float32),
            pltpu.VMEM((B, HH), jnp.float32),
            pltpu.VMEM((B, HH), jnp.float32),
            pltpu.VMEM((B, HH), jnp.float32),
            pltpu.VMEM((ZR, HH), jnp.float32),
            pltpu.VMEM_SHARED((NP, HH), jnp.float32),
        ] + [pltpu.SemaphoreType.DMA] * 12,
        compiler_params=pltpu.CompilerParams(
            use_tc_tiling_on_sc=False, needs_layout_passes=False),
    )


_CHUNKS = (20, 28, 36, 41)


_OSTORE = [(sp // 32) * 32 + 2 * (sp % 16) + (sp % 32) // 16 for sp in range(H)]


def kernel(x, edge_index, edge_length, edge_attr,
           nn0_w, nn0_b, nn2_w, nn2_b, lin1_w, lin2_w, lin2_b, lin_w, lin_b):
    ei = edge_index.astype(jnp.int32)
    el128 = edge_length.reshape(E // _BE, _BE // 128, 128)
    eat = edge_attr.T
    w0t = nn0_w.T
    b0 = nn0_b.reshape(1, H)
    ost = jnp.asarray(_OSTORE, jnp.int32)
    w2t = nn2_w.T[:, ost]
    b2 = nn2_b[ost].reshape(1, H)

    hcat = _lin1(x, lin1_w.T)
    hb = hcat.astype(jnp.bfloat16)
    h32 = lax.bitcast_convert_type(
        hb.reshape(2 * NP, HH // 2, 2), jnp.int32)
    parts = []
    blk_off = 0
    w = None
    for nblk in _CHUNKS:
        e0 = blk_off * _BE
        e1 = e0 + nblk * _BE
        nb = (e1 - e0) // NS // B
        src3 = ei[0, e0:e1].reshape(NS, nb, B)
        dst3 = ei[1, e0:e1].reshape(NS, nb, B)
        b0c = b0 if w is None else b0 + w[0, 0] * 0.0
        w = _edge_filter(eat, el128, w0t, b0c, w2t, b2, nblk, blk_off)
        parts.append(_sc_msgpass_call(nb, 0)(h32, w, src3, dst3))
        blk_off += nblk
    return _tail(parts, lin2_w.T[ost, :], lin2_b.reshape(1, H),
                 lin_w.T, lin_b.reshape(1, H))


# speedup vs baseline: 1.4088x; 1.4088x over previous
import functools

import jax
import jax.numpy as jnp
from jax import lax
from jax.experimental import pallas as pl
from jax.experimental.pallas import tpu as pltpu
from jax.experimental.pallas import tpu_sc as plsc

N, E, H, G = 10000, 320000, 128, 16
NP = 10240
HH = H // 2
NC, NS = 2, 16
EPW = E // NS
B = 80
NB = EPW // B
RPT = NP // NS
ZR = 128

_LOG2 = 0.6931471805599453
_PI = 3.141592653589793
_CUT = 10.0


def _ssp(v):
    return jnp.maximum(v, 0.0) + jnp.log1p(jnp.exp(-jnp.abs(v))) - _LOG2


_BE = 2560


def _filter_body(eat, el, w0, b0, w2, b2, out):
    a = eat[...]
    z = lax.dot_general(a, w0[...], (((0,), (0,)), ((), ())),
                        preferred_element_type=jnp.float32) + b0[...]
    w = jnp.dot(_ssp(z), w2[...], preferred_element_type=jnp.float32) + b2[...]
    l = el[0]
    env = 0.5 * (jnp.cos(l * (_PI / _CUT)) + 1.0)
    env = jnp.where((l <= _CUT) & (l >= 0.0), env, 0.0)
    w3 = w.reshape(_BE // 128, 128, H) * env[:, :, None]
    out[...] = w3.reshape(_BE, H)


def _edge_filter(edge_attr_t, el128, w0t, b0, w2t, b2, nblk, blk_off):
    be = _BE
    return pl.pallas_call(
        _filter_body,
        grid=(nblk,),
        in_specs=[
            pl.BlockSpec((G, be), lambda i: (0, blk_off + i)),
            pl.BlockSpec((1, be // 128, 128), lambda i: (blk_off + i, 0, 0)),
            pl.BlockSpec((G, H), lambda i: (0, 0)),
            pl.BlockSpec((1, H), lambda i: (0, 0)),
            pl.BlockSpec((H, H), lambda i: (0, 0)),
            pl.BlockSpec((1, H), lambda i: (0, 0)),
        ],
        out_specs=pl.BlockSpec((be, H), lambda i: (i, 0)),
        out_shape=jax.ShapeDtypeStruct((nblk * be, H), jnp.float32),
    )(edge_attr_t, el128, w0t, b0, w2t, b2)


def _lin1_body(xr, wr, out):
    h = jnp.dot(xr[...], wr[...], preferred_element_type=jnp.float32)
    out[pl.ds(0, N), :] = h[:, :HH]
    out[pl.ds(NP, N), :] = h[:, HH:]


def _lin1(x, w1t):
    return pl.pallas_call(
        _lin1_body,
        out_shape=jax.ShapeDtypeStruct((2 * NP, HH), jnp.float32),
    )(x, w1t)


def _tail_body(p0, p1, p2, p3, w2l, b2l, wl, bl, out):
    acc = p0[...] + p1[...] + p2[...] + p3[...]
    r = jnp.concatenate([acc[0, :, :HH], acc[1, :, :HH]], axis=1)
    t = _ssp(jnp.dot(r, w2l[...], preferred_element_type=jnp.float32) + b2l[...])
    out[...] = jnp.dot(t, wl[...], preferred_element_type=jnp.float32) + bl[...]


def _tail(parts, w2lt, b2l, wlt, bl):
    bn = 2000
    pspec = pl.BlockSpec((NC, bn, H), lambda i: (0, i, 0))
    return pl.pallas_call(
        _tail_body,
        grid=(N // bn,),
        in_specs=[
            pspec, pspec, pspec, pspec,
            pl.BlockSpec((H, H), lambda i: (0, 0)),
            pl.BlockSpec((1, H), lambda i: (0, 0)),
            pl.BlockSpec((H, H), lambda i: (0, 0)),
            pl.BlockSpec((1, H), lambda i: (0, 0)),
        ],
        out_specs=pl.BlockSpec((bn, H), lambda i: (i, 0)),
        out_shape=jax.ShapeDtypeStruct((N, H), jnp.float32),
    )(*parts, w2lt, b2l, wlt, bl)


@functools.cache
def _sc_msgpass_call(nb, chunk_base):
    epw = nb * B

    def _sc_msgpass(h_hbm, w_hbm, src_hbm, dst_hbm, out_hbm,
                    src_v, dst_v, gbuf_a, gbuf_b, gbuf_c, gbuf_d,
                    wbuf_a, wbuf_b, wbuf_c, wbuf_d, zbuf, acc,
                    sem_ga, sem_gb, sem_gc, sem_gd,
                    sem_wa, sem_wb, sem_wc, sem_wd,
                    sem_sa, sem_sb, sem_sc, sem_sd):
        c = lax.axis_index("c")
        s = lax.axis_index("s")

        pltpu.sync_copy(src_hbm.at[s], src_v)
        pltpu.sync_copy(dst_hbm.at[s], dst_v)

        coff = jnp.full((16,), c * NP, jnp.int32)

        def orow(r, carry):
            for k in range(B // 16):
                sl = pl.ds(k * 16, 16)
                src_v[r, sl] = src_v[r, sl] + coff
            return carry

        lax.fori_loop(0, nb, orow, 0)

        zeros = jnp.zeros((16,), jnp.float32)

        def zrow(r, carry):
            for k in range(HH // 16):
                zbuf[r, pl.ds(k * 16, 16)] = zeros
            return carry

        lax.fori_loop(0, ZR, zrow, 0)

        def zcopy(t, carry):
            pltpu.sync_copy(zbuf, acc.at[pl.ds(s * RPT + t * ZR, ZR)])
            return carry

        lax.fori_loop(0, RPT // ZR, zcopy, 0)
        plsc.subcore_barrier()

        ebase = chunk_base + s * epw

        def fetch(j, gbuf, wbuf, sem_g, sem_w):
            pltpu.async_copy(h_hbm.at[src_v.at[j]], gbuf, sem_g)
            pltpu.async_copy(
                w_hbm.at[pl.ds(ebase + j * B, B), pl.ds(c * HH, HH)],
                wbuf, sem_w)

        def wait_fetch(j, gbuf, wbuf, sem_g, sem_w):
            pltpu.make_async_copy(h_hbm.at[src_v.at[j]], gbuf, sem_g).wait()
            pltpu.make_async_copy(
                w_hbm.at[pl.ds(ebase + j * B, B), pl.ds(c * HH, HH)],
                wbuf, sem_w).wait()

        def mul(gbuf, wbuf):
            def mrow(r4, inner):
                for dr in range(4):
                    for k in range(HH // 16):
                        sl = pl.ds(k * 16, 16)
                        r = r4 * 4 + dr
                        gbuf[r, sl] = gbuf[r, sl] * wbuf[r, sl]
                return inner
            lax.fori_loop(0, B // 4, mrow, 0)

        def scat(j, gbuf, sem_s):
            pltpu.async_copy(gbuf, acc.at[dst_v.at[j]], sem_s, add=True)

        def wait_scat(j, gbuf, sem_s):
            pltpu.make_async_copy(gbuf, acc.at[dst_v.at[j]], sem_s).wait()

        K = 4
        gbufs = [gbuf_a, gbuf_b, gbuf_c, gbuf_d]
        wbufs = [wbuf_a, wbuf_b, wbuf_c, wbuf_d]
        sgs = [sem_ga, sem_gb, sem_gc, sem_gd]
        sws = [sem_wa, sem_wb, sem_wc, sem_wd]
        sss = [sem_sa, sem_sb, sem_sc, sem_sd]

        for t in range(K - 1):
            fetch(t, gbufs[t], wbufs[t], sgs[t], sws[t])

        ngrp = nb // K
        rem = nb - ngrp * K

        def step(j, t, drain):
            wait_fetch(j, gbufs[t], wbufs[t], sgs[t], sws[t])
            mul(gbufs[t], wbufs[t])
            scat(j, gbufs[t], sss[t])
            tf = (t - 1) % K
            jf = j + K - 1

            @pl.when(jf < nb)
            def _():
                if drain:
                    wait_scat(j - 1, gbufs[tf], sss[tf])
                fetch(jf, gbufs[tf], wbufs[tf], sgs[tf], sws[tf])

        for t in range(K):
            step(t, t, t > 0)

        def body(g, carry):
            j0 = g * K
            for t in range(K):
                step(j0 + t, t, True)
            return carry

        lax.fori_loop(1, ngrp, body, 0)
        for r in range(rem):
            step(ngrp * K + r, r, True)
        for r in range(K):
            j = nb - K + r
            wait_scat(j, gbufs[(nb - K + r) % K], sss[(nb - K + r) % K])

        plsc.subcore_barrier()
        pltpu.sync_copy(acc.at[pl.ds(s * RPT, RPT)],
                        out_hbm.at[c, pl.ds(s * RPT, RPT), pl.ds(0, HH)])

    mesh = plsc.VectorSubcoreMesh(
        core_axis_name="c", subcore_axis_name="s",
        num_cores=NC, num_subcores=NS)
    return pl.kernel(
        _sc_msgpass,
        out_type=jax.ShapeDtypeStruct((NC, NP, H), jnp.float32),
        mesh=mesh,
        scratch_types=[
            pltpu.VMEM((nb, B), jnp.int32),
            pltpu.VMEM((nb, B), jnp.int32),
            pltpu.VMEM((B, HH), jnp.float32),
            pltpu.VMEM((B, HH), jnp.float32),
            pltpu.VMEM((B, HH), jnp.float32),
            pltpu.VMEM((B, HH), jnp.float32),
            pltpu.VMEM((B, HH), jnp.float32),
            pltpu.VMEM((B, HH), jnp.float32),
            pltpu.VMEM((B, HH), jnp.float32),
            pltpu.VMEM((B, HH), jnp.float32),
            pltpu.VMEM((ZR, HH), jnp.float32),
            pltpu.VMEM_SHARED((NP, HH), jnp.float32),
        ] + [pltpu.SemaphoreType.DMA] * 12,
        compiler_params=pltpu.CompilerParams(use_tc_tiling_on_sc=False),
    )


_CHUNKS = (20, 28, 36, 41)


def kernel(x, edge_index, edge_length, edge_attr,
           nn0_w, nn0_b, nn2_w, nn2_b, lin1_w, lin2_w, lin2_b, lin_w, lin_b):
    ei = edge_index.astype(jnp.int32)
    el128 = edge_length.reshape(E // _BE, _BE // 128, 128)
    eat = edge_attr.T
    w0t = nn0_w.T
    b0 = nn0_b.reshape(1, H)
    w2t = nn2_w.T
    b2 = nn2_b.reshape(1, H)

    hcat = _lin1(x, lin1_w.T)
    parts = []
    blk_off = 0
    w = None
    for nblk in _CHUNKS:
        e0 = blk_off * _BE
        e1 = e0 + nblk * _BE
        nb = (e1 - e0) // NS // B
        src3 = ei[0, e0:e1].reshape(NS, nb, B)
        dst3 = ei[1, e0:e1].reshape(NS, nb, B)
        b0c = b0 if w is None else b0 + w[0, 0] * 0.0
        w = _edge_filter(eat, el128, w0t, b0c, w2t, b2, nblk, blk_off)
        parts.append(_sc_msgpass_call(nb, 0)(hcat, w, src3, dst3))
        blk_off += nblk
    return _tail(parts, lin2_w.T, lin2_b.reshape(1, H),
                 lin_w.T, lin_b.reshape(1, H))


# speedup vs baseline: 1.4375x; 1.0204x over previous
import functools

import jax
import jax.numpy as jnp
from jax import lax
from jax.experimental import pallas as pl
from jax.experimental.pallas import tpu as pltpu
from jax.experimental.pallas import tpu_sc as plsc

N, E, H, G = 10000, 320000, 128, 16
NP = 10240
HH = H // 2
NC, NS = 2, 16
EPW = E // NS
B = 80
NB = EPW // B
RPT = NP // NS
ZR = 128

_LOG2 = 0.6931471805599453
_PI = 3.141592653589793
_CUT = 10.0


def _ssp(v):
    return jnp.maximum(v, 0.0) + jnp.log1p(jnp.exp(-jnp.abs(v))) - _LOG2


_BE = 2560


def _filter_body(eat, el, w0, b0, w2, b2, out):
    a = eat[...]
    z = lax.dot_general(a, w0[...], (((0,), (0,)), ((), ())),
                        preferred_element_type=jnp.float32) + b0[...]
    s = jnp.log(0.5 + 0.5 * jnp.exp(z))
    w = jnp.dot(s, w2[...], preferred_element_type=jnp.float32,
                precision=lax.Precision.DEFAULT) + b2[...]
    l = el[0]
    env = 0.5 * (jnp.cos(l * (_PI / _CUT)) + 1.0)
    env = jnp.where((l <= _CUT) & (l >= 0.0), env, 0.0)
    w3 = w.reshape(_BE // 128, 128, H) * env[:, :, None]
    out[...] = w3.reshape(_BE, H)


def _edge_filter(edge_attr_t, el128, w0t, b0, w2t, b2, nblk, blk_off):
    be = _BE
    return pl.pallas_call(
        _filter_body,
        grid=(nblk,),
        in_specs=[
            pl.BlockSpec((G, be), lambda i: (0, blk_off + i)),
            pl.BlockSpec((1, be // 128, 128), lambda i: (blk_off + i, 0, 0)),
            pl.BlockSpec((G, H), lambda i: (0, 0)),
            pl.BlockSpec((1, H), lambda i: (0, 0)),
            pl.BlockSpec((H, H), lambda i: (0, 0)),
            pl.BlockSpec((1, H), lambda i: (0, 0)),
        ],
        out_specs=pl.BlockSpec((be, H), lambda i: (i, 0)),
        out_shape=jax.ShapeDtypeStruct((nblk * be, H), jnp.float32),
    )(edge_attr_t, el128, w0t, b0, w2t, b2)


def _lin1_body(xr, wr, out):
    h = jnp.dot(xr[...], wr[...], preferred_element_type=jnp.float32)
    out[pl.ds(0, N), :] = h[:, :HH]
    out[pl.ds(NP, N), :] = h[:, HH:]


def _lin1(x, w1t):
    return pl.pallas_call(
        _lin1_body,
        out_shape=jax.ShapeDtypeStruct((2 * NP, HH), jnp.float32),
    )(x, w1t)


def _tail_body(p0, p1, p2, p3, w2l, b2l, wl, bl, out):
    acc = p0[...] + p1[...] + p2[...] + p3[...]
    r = jnp.concatenate([acc[0, :, :HH], acc[1, :, :HH]], axis=1)
    t = _ssp(jnp.dot(r, w2l[...], preferred_element_type=jnp.float32) + b2l[...])
    out[...] = jnp.dot(t, wl[...], preferred_element_type=jnp.float32) + bl[...]


def _tail(parts, w2lt, b2l, wlt, bl):
    bn = 2000
    pspec = pl.BlockSpec((NC, bn, H), lambda i: (0, i, 0))
    return pl.pallas_call(
        _tail_body,
        grid=(N // bn,),
        in_specs=[
            pspec, pspec, pspec, pspec,
            pl.BlockSpec((H, H), lambda i: (0, 0)),
            pl.BlockSpec((1, H), lambda i: (0, 0)),
            pl.BlockSpec((H, H), lambda i: (0, 0)),
            pl.BlockSpec((1, H), lambda i: (0, 0)),
        ],
        out_specs=pl.BlockSpec((bn, H), lambda i: (i, 0)),
        out_shape=jax.ShapeDtypeStruct((N, H), jnp.float32),
    )(*parts, w2lt, b2l, wlt, bl)


@functools.cache
def _sc_msgpass_call(nb, chunk_base):
    epw = nb * B

    def _sc_msgpass(h_hbm, w_hbm, src_hbm, dst_hbm, out_hbm,
                    src_v, dst_v, gbuf_a, gbuf_b, gbuf_c, gbuf_d,
                    wbuf_a, wbuf_b, wbuf_c, wbuf_d, zbuf, acc,
                    sem_ga, sem_gb, sem_gc, sem_gd,
                    sem_wa, sem_wb, sem_wc, sem_wd,
                    sem_sa, sem_sb, sem_sc, sem_sd):
        c = lax.axis_index("c")
        s = lax.axis_index("s")

        pltpu.sync_copy(src_hbm.at[s], src_v)
        pltpu.sync_copy(dst_hbm.at[s], dst_v)

        coff = jnp.full((16,), c * NP, jnp.int32)

        def orow(r, carry):
            for k in range(B // 16):
                sl = pl.ds(k * 16, 16)
                src_v[r, sl] = src_v[r, sl] + coff
            return carry

        lax.fori_loop(0, nb, orow, 0)

        zeros = jnp.zeros((16,), jnp.float32)

        def zrow(r, carry):
            for k in range(HH // 16):
                zbuf[r, pl.ds(k * 16, 16)] = zeros
            return carry

        lax.fori_loop(0, ZR, zrow, 0)

        def zcopy(t, carry):
            pltpu.sync_copy(zbuf, acc.at[pl.ds(s * RPT + t * ZR, ZR)])
            return carry

        lax.fori_loop(0, RPT // ZR, zcopy, 0)
        plsc.subcore_barrier()

        ebase = chunk_base + s * epw

        def fetch(j, gbuf, wbuf, sem_g, sem_w):
            pltpu.async_copy(h_hbm.at[src_v.at[j]], gbuf, sem_g)
            pltpu.async_copy(
                w_hbm.at[pl.ds(ebase + j * B, B), pl.ds(c * HH, HH)],
                wbuf, sem_w)

        def wait_fetch(j, gbuf, wbuf, sem_g, sem_w):
            pltpu.make_async_copy(h_hbm.at[src_v.at[j]], gbuf, sem_g).wait()
            pltpu.make_async_copy(
                w_hbm.at[pl.ds(ebase + j * B, B), pl.ds(c * HH, HH)],
                wbuf, sem_w).wait()

        def mul(gbuf, wbuf):
            def mrow(r4, inner):
                for dr in range(4):
                    for k in range(HH // 16):
                        sl = pl.ds(k * 16, 16)
                        r = r4 * 4 + dr
                        gbuf[r, sl] = gbuf[r, sl] * wbuf[r, sl]
                return inner
            lax.fori_loop(0, B // 4, mrow, 0)

        def scat(j, gbuf, sem_s):
            pltpu.async_copy(gbuf, acc.at[dst_v.at[j]], sem_s, add=True)

        def wait_scat(j, gbuf, sem_s):
            pltpu.make_async_copy(gbuf, acc.at[dst_v.at[j]], sem_s).wait()

        K = 4
        gbufs = [gbuf_a, gbuf_b, gbuf_c, gbuf_d]
        wbufs = [wbuf_a, wbuf_b, wbuf_c, wbuf_d]
        sgs = [sem_ga, sem_gb, sem_gc, sem_gd]
        sws = [sem_wa, sem_wb, sem_wc, sem_wd]
        sss = [sem_sa, sem_sb, sem_sc, sem_sd]

        for t in range(K - 1):
            fetch(t, gbufs[t], wbufs[t], sgs[t], sws[t])

        ngrp = nb // K
        rem = nb - ngrp * K

        def step(j, t, drain):
            wait_fetch(j, gbufs[t], wbufs[t], sgs[t], sws[t])
            mul(gbufs[t], wbufs[t])
            scat(j, gbufs[t], sss[t])
            tf = (t - 1) % K
            jf = j + K - 1

            @pl.when(jf < nb)
            def _():
                if drain:
                    wait_scat(j - 1, gbufs[tf], sss[tf])
                fetch(jf, gbufs[tf], wbufs[tf], sgs[tf], sws[tf])

        for t in range(K):
            step(t, t, t > 0)

        def body(g, carry):
            j0 = g * K
            for t in range(K):
                step(j0 + t, t, True)
            return carry

        lax.fori_loop(1, ngrp, body, 0)
        for r in range(rem):
            step(ngrp * K + r, r, True)
        for r in range(K):
            j = nb - K + r
            wait_scat(j, gbufs[(nb - K + r) % K], sss[(nb - K + r) % K])

        plsc.subcore_barrier()
        pltpu.sync_copy(acc.at[pl.ds(s * RPT, RPT)],
                        out_hbm.at[c, pl.ds(s * RPT, RPT), pl.ds(0, HH)])

    mesh = plsc.VectorSubcoreMesh(
        core_axis_name="c", subcore_axis_name="s",
        num_cores=NC, num_subcores=NS)
    return pl.kernel(
        _sc_msgpass,
        out_type=jax.ShapeDtypeStruct((NC, NP, H), jnp.float32),
        mesh=mesh,
        scratch_types=[
            pltpu.VMEM((nb, B), jnp.int32),
            pltpu.VMEM((nb, B), jnp.int32),
            pltpu.VMEM((B, HH), jnp.float32),
            pltpu.VMEM((B, HH), jnp.float32),
            pltpu.VMEM((B, HH), jnp.float32),
            pltpu.VMEM((B, HH), jnp.float32),
            pltpu.VMEM((B, HH), jnp.float32),
            pltpu.VMEM((B, HH), jnp.float32),
            pltpu.VMEM((B, HH), jnp.float32),
            pltpu.VMEM((B, HH), jnp.float32),
            pltpu.VMEM((ZR, HH), jnp.float32),
            pltpu.VMEM_SHARED((NP, HH), jnp.float32),
        ] + [pltpu.SemaphoreType.DMA] * 12,
        compiler_params=pltpu.CompilerParams(use_tc_tiling_on_sc=False),
    )


_CHUNKS = (20, 28, 36, 41)


def kernel(x, edge_index, edge_length, edge_attr,
           nn0_w, nn0_b, nn2_w, nn2_b, lin1_w, lin2_w, lin2_b, lin_w, lin_b):
    ei = edge_index.astype(jnp.int32)
    el128 = edge_length.reshape(E // _BE, _BE // 128, 128)
    eat = edge_attr.T
    w0t = nn0_w.T
    b0 = nn0_b.reshape(1, H)
    w2t = nn2_w.T
    b2 = nn2_b.reshape(1, H)

    hcat = _lin1(x, lin1_w.T)
    parts = []
    blk_off = 0
    w = None
    for nblk in _CHUNKS:
        e0 = blk_off * _BE
        e1 = e0 + nblk * _BE
        nb = (e1 - e0) // NS // B
        src3 = ei[0, e0:e1].reshape(NS, nb, B)
        dst3 = ei[1, e0:e1].reshape(NS, nb, B)
        b0c = b0 if w is None else b0 + w[0, 0] * 0.0
        w = _edge_filter(eat, el128, w0t, b0c, w2t, b2, nblk, blk_off)
        parts.append(_sc_msgpass_call(nb, 0)(hcat, w, src3, dst3))
        blk_off += nblk
    return _tail(parts, lin2_w.T, lin2_b.reshape(1, H),
                 lin_w.T, lin_b.reshape(1, H))
